# Initial kernel scaffold; baseline (speedup 1.0000x reference)
import jax
import jax.numpy as jnp
from jax.experimental import pallas as pl


def kernel(x, edge_index, edge_vals, W):
    raise NotImplementedError("write your pallas kernel here")


# speedup vs baseline: 4.3853x; 4.3853x over previous
import functools

import jax
import jax.numpy as jnp
from jax import lax
from jax.experimental import pallas as pl
from jax.experimental.pallas import tpu as pltpu
from jax.experimental.pallas import tpu_sc as plsc

NC = 2
NS = 16
NW = NC * NS
CHUNK = 128
LANES = 16


def _spmm_sc(x, col3, row3, val3, n_chunks, n_nodes, d):
    rows_per_tile = n_nodes // NS
    n_full = rows_per_tile // CHUNK
    rem = rows_per_tile % CHUNK
    mesh = plsc.VectorSubcoreMesh(core_axis_name="c", subcore_axis_name="s")

    @functools.partial(
        pl.kernel,
        mesh=mesh,
        out_type=jax.ShapeDtypeStruct((NC, n_nodes, d), jnp.float32),
        scratch_types=[
            pltpu.VMEM((n_chunks, CHUNK), jnp.int32),
            pltpu.VMEM((n_chunks, CHUNK), jnp.int32),
            pltpu.VMEM((n_chunks, CHUNK), jnp.float32),
            pltpu.VMEM((CHUNK, d), jnp.float32),
            pltpu.VMEM_SHARED((n_nodes, d), jnp.float32),
            pltpu.SemaphoreType.DMA,
        ],
    )
    def spmm(x_hbm, col_hbm, row_hbm, val_hbm, out_hbm,
             colbuf, rowbuf, valbuf, rows, acc, sem):
        cid = lax.axis_index("c")
        sid = lax.axis_index("s")
        wid = sid * NC + cid

        pltpu.sync_copy(col_hbm.at[wid], colbuf)
        pltpu.sync_copy(row_hbm.at[wid], rowbuf)
        pltpu.sync_copy(val_hbm.at[wid], valbuf)

        def zero_body(e, _):
            for s in range(d // LANES):
                rows[e, pl.ds(s * LANES, LANES)] = jnp.zeros(
                    (LANES,), jnp.float32)
            return 0
        lax.fori_loop(0, CHUNK, zero_body, 0)

        base = sid * rows_per_tile
        for b in range(n_full):
            pltpu.sync_copy(rows, acc.at[pl.ds(base + b * CHUNK, CHUNK)])
        if rem:
            pltpu.sync_copy(rows.at[pl.ds(0, rem)],
                            acc.at[pl.ds(base + n_full * CHUNK, rem)])
        plsc.subcore_barrier()

        def chunk_body(c, _):
            pltpu.async_copy(x_hbm.at[colbuf.at[c]], rows, sem).wait()

            def scale_body(g, _):
                vg = valbuf[c, pl.ds(g * LANES, LANES)]
                for j in range(LANES):
                    e = g * LANES + j
                    v = vg[j]
                    for s in range(d // LANES):
                        sl = pl.ds(s * LANES, LANES)
                        rows[e, sl] = rows[e, sl] * v
                return 0
            lax.fori_loop(0, CHUNK // LANES, scale_body, 0)

            pltpu.sync_copy(rows, acc.at[rowbuf.at[c]], add=True)
            return 0
        lax.fori_loop(0, n_chunks, chunk_body, 0)
        plsc.subcore_barrier()

        pltpu.sync_copy(acc.at[pl.ds(base, rows_per_tile)],
                        out_hbm.at[cid, pl.ds(base, rows_per_tile)])

    return spmm(x, col3, row3, val3)


def _finish_tc(partial, W, n_nodes, d):
    blk = 1024

    def body(p_ref, w_ref, o_ref):
        acc = p_ref[0] + p_ref[1]
        o_ref[...] = jnp.dot(acc, w_ref[...],
                             preferred_element_type=jnp.float32)

    return pl.pallas_call(
        body,
        grid=(n_nodes // blk,),
        in_specs=[
            pl.BlockSpec((2, blk, d), lambda i: (0, i, 0)),
            pl.BlockSpec((d, d), lambda i: (0, 0)),
        ],
        out_specs=pl.BlockSpec((blk, d), lambda i: (i, 0)),
        out_shape=jax.ShapeDtypeStruct((n_nodes, d), jnp.float32),
    )(partial, W)


def kernel(x, edge_index, edge_vals, W):
    n_nodes, d = x.shape
    n_pad = -(-n_nodes // (NS * CHUNK)) * (NS * CHUNK)
    row = edge_index[0].astype(jnp.int32)
    col = edge_index[1].astype(jnp.int32)
    vals = edge_vals.astype(jnp.float32)

    e = row.shape[0]
    per_tile = -(-e // NW)
    n_chunks = -(-per_tile // CHUNK)
    e_pad = n_chunks * CHUNK * NW
    pad = e_pad - e
    row = jnp.pad(row, (0, pad)).reshape(NW, n_chunks, CHUNK)
    col = jnp.pad(col, (0, pad)).reshape(NW, n_chunks, CHUNK)
    vals = jnp.pad(vals, (0, pad)).reshape(NW, n_chunks, CHUNK)

    partial = _spmm_sc(x, col, row, vals, n_chunks, n_pad, d)
    return _finish_tc(partial, W, n_pad, d)[:n_nodes]
